# baseline (speedup 1.0000x reference)
import jax
import jax.numpy as jnp
from jax.experimental import pallas as pl
from jax.experimental.pallas import tpu as pltpu

_M = 16 * 576
_K = 64
_C = 1024
_M_BLK = 1024


def _assign_kernel(x_ref, c_ref, out_ref):
    x = x_ref[...]
    c = c_ref[...]
    dot = jax.lax.dot_general(
        x, c, (((1,), (1,)), ((), ())),
        preferred_element_type=jnp.float32,
    )
    nx = jnp.sum(x * x, axis=1, keepdims=True)
    nc = jnp.sum(c * c, axis=1)[None, :]
    d = (nx + nc) - 2.0 * dot
    m = jnp.min(d, axis=1, keepdims=True)
    ids = jax.lax.broadcasted_iota(jnp.int32, d.shape, 1)
    idx = jnp.min(jnp.where(d == m, ids, jnp.int32(_C)), axis=1)
    out_ref[...] = idx


def kernel(x, centroids):
    xf = x.reshape(_M, _K)
    out = pl.pallas_call(
        _assign_kernel,
        grid=(_M // _M_BLK,),
        in_specs=[
            pl.BlockSpec((_M_BLK, _K), lambda i: (i, 0)),
            pl.BlockSpec((_C, _K), lambda i: (0, 0)),
        ],
        out_specs=pl.BlockSpec((_M_BLK,), lambda i: (i,)),
        out_shape=jax.ShapeDtypeStruct((_M,), jnp.int32),
    )(xf, centroids)
    return out.reshape(x.shape[:2])


# speedup vs baseline: 1.2085x; 1.2085x over previous
import jax
import jax.numpy as jnp
from jax.experimental import pallas as pl
from jax.experimental.pallas import tpu as pltpu

_M = 16 * 576
_K = 64
_C = 1024
_G = 128
_M_BLK = 3072
_BIG = 3.0e38


def _assign_kernel(x_ref, c_ref, out_ref):
    x = x_ref[...]
    c = c_ref[...]
    nx = jnp.sum(x * x, axis=1, keepdims=True)
    nc = jnp.sum(c * c, axis=1)[None, :]
    x2 = x * 2.0

    runmin = jnp.full((_M_BLK, _G), _BIG, jnp.float32)
    rung = jnp.zeros((_M_BLK, _G), jnp.int32)
    for g in range(_C // _G):
        cg = c[g * _G:(g + 1) * _G, :]
        dot2 = jax.lax.dot_general(
            x2, cg, (((1,), (1,)), ((), ())),
            preferred_element_type=jnp.float32,
        )
        d = (nx + nc[:, g * _G:(g + 1) * _G]) - dot2
        mask = d < runmin
        rung = jnp.where(mask, jnp.int32(g), rung)
        runmin = jnp.minimum(runmin, d)

    m = jnp.min(runmin, axis=1, keepdims=True)
    lane = jax.lax.broadcasted_iota(jnp.int32, (_M_BLK, _G), 1)
    idxfull = rung * _G + lane
    cand = jnp.where(runmin == m, idxfull, jnp.int32(_C))
    out_ref[...] = jnp.min(cand, axis=1)


def kernel(x, centroids):
    xf = x.reshape(_M, _K)
    out = pl.pallas_call(
        _assign_kernel,
        grid=(_M // _M_BLK,),
        in_specs=[
            pl.BlockSpec((_M_BLK, _K), lambda i: (i, 0)),
            pl.BlockSpec((_C, _K), lambda i: (0, 0)),
        ],
        out_specs=pl.BlockSpec((_M_BLK,), lambda i: (i,)),
        out_shape=jax.ShapeDtypeStruct((_M,), jnp.int32),
    )(xf, centroids)
    return out.reshape(x.shape[:2])


# speedup vs baseline: 1.6119x; 1.3338x over previous
import jax
import jax.numpy as jnp
from jax.experimental import pallas as pl
from jax.experimental.pallas import tpu as pltpu

_B = 16
_R = 576
_K = 64
_C = 1024
_G = 128
_NG = _C // _G
_B_BLK = 8
_M = _B_BLK * _R
_BIG = 3.0e38


def _chain(dot2, nx, nc, groups):
    runmin = jnp.full((_M, _G), _BIG, jnp.float32)
    rung = jnp.zeros((_M, _G), jnp.int32)
    for g in groups:
        d = (nx + nc[:, g * _G:(g + 1) * _G]) - dot2[:, g * _G:(g + 1) * _G]
        mask = d < runmin
        rung = jnp.where(mask, jnp.int32(g), rung)
        runmin = jnp.minimum(runmin, d)
    return runmin, rung


def _assign_kernel(x_ref, c_ref, out_ref):
    x = x_ref[...].reshape(_M, _K)
    c = c_ref[...]
    nx = jnp.sum(x * x, axis=1, keepdims=True)
    nc = jnp.sum(c * c, axis=1)[None, :]
    dot2 = jax.lax.dot_general(
        x * 2.0, c, (((1,), (1,)), ((), ())),
        preferred_element_type=jnp.float32,
    )

    ra, ga = _chain(dot2, nx, nc, range(_NG // 2))
    rb, gb = _chain(dot2, nx, nc, range(_NG // 2, _NG))
    swap = rb < ra
    runmin = jnp.minimum(ra, rb)
    rung = jnp.where(swap, gb, ga)

    m = jnp.min(runmin, axis=1, keepdims=True)
    lane = jax.lax.broadcasted_iota(jnp.int32, (_M, _G), 1)
    idxfull = rung * _G + lane
    cand = jnp.where(runmin == m, idxfull, jnp.int32(_C))
    idx = jnp.min(cand.reshape(_B_BLK, _R, _G), axis=2)
    out_ref[...] = idx


def kernel(x, centroids):
    out = pl.pallas_call(
        _assign_kernel,
        grid=(_B // _B_BLK,),
        in_specs=[
            pl.BlockSpec((_B_BLK, _R, _K), lambda i: (i, 0, 0)),
            pl.BlockSpec((_C, _K), lambda i: (0, 0)),
        ],
        out_specs=pl.BlockSpec((_B_BLK, _R), lambda i: (i, 0)),
        out_shape=jax.ShapeDtypeStruct((_B, _R), jnp.int32),
    )(x, centroids)
    return out
